# Initial kernel scaffold; baseline (speedup 1.0000x reference)
import functools

import jax
import jax.numpy as jnp
from jax import lax
from jax.experimental import pallas as pl
from jax.experimental.pallas import tpu as pltpu
from jax.experimental.pallas import tpu_sc as plsc

NC = 2
NS = 16
NW = NC * NS
CB = 128

DEGW = 16

_MESH = plsc.VectorSubcoreMesh(
    core_axis_name="c", subcore_axis_name="s", num_cores=NC, num_subcores=NS
)


def _sc_degree_fn(npad, nchunk):
    rows_per_tile = npad // NS

    @functools.partial(
        pl.kernel,
        out_type=jax.ShapeDtypeStruct((NC, npad, DEGW), jnp.float32),
        mesh=_MESH,
        scratch_types=[
            pltpu.VMEM((nchunk, CB), jnp.int32),
            pltpu.VMEM((CB, DEGW), jnp.float32),
            pltpu.VMEM_SHARED((npad, DEGW), jnp.float32),
        ],
    )
    def deg_kernel(dst_hbm, ones_hbm, zeros_hbm, out_hbm, dstv, onesv, acc):
        c = lax.axis_index("c")
        s = lax.axis_index("s")
        wid = s * NC + c
        r0 = s * rows_per_tile
        pltpu.sync_copy(zeros_hbm.at[pl.ds(r0, rows_per_tile)],
                        acc.at[pl.ds(r0, rows_per_tile)])
        pltpu.sync_copy(ones_hbm, onesv)
        pltpu.sync_copy(dst_hbm.at[wid], dstv)
        plsc.subcore_barrier()

        def body(j, carry):
            pltpu.sync_copy(onesv, acc.at[dstv.at[j]], add=True)
            return carry

        lax.fori_loop(0, nchunk, body, 0)
        plsc.subcore_barrier()
        pltpu.sync_copy(acc.at[pl.ds(r0, rows_per_tile)],
                        out_hbm.at[c, pl.ds(r0, rows_per_tile)])

    return deg_kernel


def _sc_scatter_fn(n, h, npad, nchunk):
    rows_per_tile = npad // NS

    @functools.partial(
        pl.kernel,
        out_type=jax.ShapeDtypeStruct((NC, npad, h), jnp.float32),
        mesh=_MESH,
        scratch_types=[
            pltpu.VMEM((nchunk, CB), jnp.int32),
            pltpu.VMEM((nchunk, CB), jnp.int32),
            pltpu.VMEM((CB, h), jnp.float32),
            pltpu.VMEM_SHARED((npad, h), jnp.float32),
            pltpu.SemaphoreType.DMA,
        ],
    )
    def scatter_kernel(g_hbm, src_hbm, dst_hbm, zeros_hbm, out_hbm,
                       srcv, dstv, rows, acc, gsem):
        c = lax.axis_index("c")
        s = lax.axis_index("s")
        wid = s * NC + c
        r0 = s * rows_per_tile
        pltpu.sync_copy(zeros_hbm.at[pl.ds(r0, rows_per_tile)],
                        acc.at[pl.ds(r0, rows_per_tile)])
        pltpu.sync_copy(src_hbm.at[wid], srcv)
        pltpu.sync_copy(dst_hbm.at[wid], dstv)
        plsc.subcore_barrier()

        def body(j, carry):
            pltpu.async_copy(g_hbm.at[srcv.at[j]], rows, gsem).wait()
            pltpu.sync_copy(rows, acc.at[dstv.at[j]], add=True)
            return carry

        lax.fori_loop(0, nchunk, body, 0)
        plsc.subcore_barrier()
        pltpu.sync_copy(acc.at[pl.ds(r0, rows_per_tile)],
                        out_hbm.at[c, pl.ds(r0, rows_per_tile)])

    return scatter_kernel


def _tc_first(degp, x, w1, br):
    n, f = x.shape
    h = w1.shape[1]
    grid = n // br

    def body(deg_ref, x_ref, w_ref, dis_ref, g_ref):
        a = deg_ref[...]
        deg = a[0, :, 0:1] + a[1, :, 0:1] + 1.0
        dis = lax.rsqrt(deg)
        m = jnp.dot(x_ref[...], w_ref[...], preferred_element_type=jnp.float32)
        g_ref[...] = dis * m
        dis_ref[...] = jnp.broadcast_to(dis, (br, h))

    return pl.pallas_call(
        body,
        grid=(grid,),
        in_specs=[
            pl.BlockSpec((NC, br, DEGW), lambda i: (0, i, 0)),
            pl.BlockSpec((br, f), lambda i: (i, 0)),
            pl.BlockSpec((f, h), lambda i: (0, 0)),
        ],
        out_specs=[
            pl.BlockSpec((br, h), lambda i: (i, 0)),
            pl.BlockSpec((br, h), lambda i: (i, 0)),
        ],
        out_shape=[
            jax.ShapeDtypeStruct((n, h), jnp.float32),
            jax.ShapeDtypeStruct((n, h), jnp.float32),
        ],
    )(degp, x, w1)


def _tc_combine(agg, g, dis64, b, w_next, br):
    n, h = g.shape
    h2 = w_next.shape[1]
    grid = n // br

    def body(agg_ref, g_ref, dis_ref, b_ref, w_ref, out_ref):
        a = agg_ref[...]
        dis = dis_ref[...]
        hcur = jax.nn.relu(dis * (a[0] + a[1] + g_ref[...]) + b_ref[...])
        out_ref[...] = dis[:, :h2] * jnp.dot(
            hcur, w_ref[...], preferred_element_type=jnp.float32)

    return pl.pallas_call(
        body,
        grid=(grid,),
        in_specs=[
            pl.BlockSpec((NC, br, h), lambda i: (0, i, 0)),
            pl.BlockSpec((br, h), lambda i: (i, 0)),
            pl.BlockSpec((br, h), lambda i: (i, 0)),
            pl.BlockSpec((1, h), lambda i: (0, 0)),
            pl.BlockSpec((h, h2), lambda i: (0, 0)),
        ],
        out_specs=pl.BlockSpec((br, h2), lambda i: (i, 0)),
        out_shape=jax.ShapeDtypeStruct((n, h2), jnp.float32),
    )(agg, g, dis64, b, w_next)


def _tc_final(agg, g, dis64, b, wfc, bfc, br):
    n, h = g.shape
    grid = n // br

    def body(agg_ref, g_ref, dis_ref, b_ref, wfc_ref, bfc_ref, out_ref, acc_ref):
        i = pl.program_id(0)
        a = agg_ref[...]
        blk = dis_ref[...] * (a[0] + a[1] + g_ref[...]) + b_ref[...]
        s = jnp.sum(blk, axis=0, keepdims=True)

        @pl.when(i == 0)
        def _():
            acc_ref[...] = s

        @pl.when(i > 0)
        def _():
            acc_ref[...] += s

        @pl.when(i == grid - 1)
        def _():
            pooled = acc_ref[...] * (1.0 / n)
            out_ref[...] = jnp.dot(
                pooled, wfc_ref[...], preferred_element_type=jnp.float32
            ) + bfc_ref[...]

    return pl.pallas_call(
        body,
        grid=(grid,),
        in_specs=[
            pl.BlockSpec((NC, br, h), lambda i: (0, i, 0)),
            pl.BlockSpec((br, h), lambda i: (i, 0)),
            pl.BlockSpec((br, h), lambda i: (i, 0)),
            pl.BlockSpec((1, h), lambda i: (0, 0)),
            pl.BlockSpec((h, 1), lambda i: (0, 0)),
            pl.BlockSpec((1, 1), lambda i: (0, 0)),
        ],
        out_specs=pl.BlockSpec((1, 1), lambda i: (0, 0)),
        out_shape=jax.ShapeDtypeStruct((1, 1), jnp.float32),
        scratch_shapes=[pltpu.VMEM((1, h), jnp.float32)],
    )(agg, g, dis64, b, wfc, bfc)


def kernel(x, edge_index, W1, b1, W2, b2, W3, b3, Wfc, bfc):
    n, f = x.shape
    h = W1.shape[1]
    e = edge_index.shape[1]

    npad = ((n + NS * CB - 1) // (NS * CB)) * (NS * CB)
    epw = ((e + NW * CB - 1) // (NW * CB)) * CB
    epad = epw * NW
    nchunk = epw // CB
    pad_row = n + 8

    src = edge_index[0]
    dst = edge_index[1]
    pad = epad - e
    src_p = jnp.concatenate(
        [src, jnp.zeros((pad,), jnp.int32)]).reshape(NW, nchunk, CB)
    dst_p = jnp.concatenate(
        [dst, jnp.full((pad,), pad_row, jnp.int32)]).reshape(NW, nchunk, CB)

    ones_deg = jnp.ones((CB, DEGW), jnp.float32)
    zeros_deg = jnp.zeros((npad, DEGW), jnp.float32)
    zeros_agg = jnp.zeros((npad, h), jnp.float32)

    br = 1000

    degp = _sc_degree_fn(npad, nchunk)(dst_p, ones_deg, zeros_deg)
    dis64, g = _tc_first(degp, x, W1, br)

    scat = _sc_scatter_fn(n, h, npad, nchunk)
    agg = scat(g, src_p, dst_p, zeros_agg)
    g = _tc_combine(agg, g, dis64, b1.reshape(1, h), W2, br)
    agg = scat(g, src_p, dst_p, zeros_agg)
    g = _tc_combine(agg, g, dis64, b2.reshape(1, h), W3, br)
    agg = scat(g, src_p, dst_p, zeros_agg)
    return _tc_final(agg, g, dis64, b3.reshape(1, h), Wfc,
                     bfc.reshape(1, 1), br)


# speedup vs baseline: 17.7293x; 17.7293x over previous
import functools

import jax
import jax.numpy as jnp
from jax import lax
from jax.experimental import pallas as pl
from jax.experimental.pallas import tpu as pltpu
from jax.experimental.pallas import tpu_sc as plsc

NC = 2
NS = 16
NW = NC * NS
CB = 128

DEGW = 16

_MESH = plsc.VectorSubcoreMesh(
    core_axis_name="c", subcore_axis_name="s", num_cores=NC, num_subcores=NS
)

_SC_PARAMS = pltpu.CompilerParams(use_tc_tiling_on_sc=False)


def _sc_degree_fn(npad, nchunk):
    rows_per_tile = npad // NS

    @functools.partial(
        pl.kernel,
        out_type=jax.ShapeDtypeStruct((NC, npad, DEGW), jnp.float32),
        mesh=_MESH,
        compiler_params=_SC_PARAMS,
        scratch_types=[
            pltpu.VMEM((nchunk, CB), jnp.int32),
            pltpu.VMEM((CB, DEGW), jnp.float32),
            pltpu.VMEM_SHARED((npad, DEGW), jnp.float32),
        ],
    )
    def deg_kernel(dst_hbm, ones_hbm, zeros_hbm, out_hbm, dstv, onesv, acc):
        c = lax.axis_index("c")
        s = lax.axis_index("s")
        wid = s * NC + c
        r0 = s * rows_per_tile
        pltpu.sync_copy(zeros_hbm.at[pl.ds(r0, rows_per_tile)],
                        acc.at[pl.ds(r0, rows_per_tile)])
        pltpu.sync_copy(ones_hbm, onesv)
        pltpu.sync_copy(dst_hbm.at[wid], dstv)
        plsc.subcore_barrier()

        def body(j, carry):
            pltpu.sync_copy(onesv, acc.at[dstv.at[j]], add=True)
            return carry

        lax.fori_loop(0, nchunk, body, 0)
        plsc.subcore_barrier()
        pltpu.sync_copy(acc.at[pl.ds(r0, rows_per_tile)],
                        out_hbm.at[c, pl.ds(r0, rows_per_tile)])

    return deg_kernel


def _sc_scatter_fn(n, h, npad, nchunk):
    rows_per_tile = npad // NS

    @functools.partial(
        pl.kernel,
        out_type=jax.ShapeDtypeStruct((NC, npad, h), jnp.float32),
        mesh=_MESH,
        compiler_params=_SC_PARAMS,
        scratch_types=[
            pltpu.VMEM((nchunk, CB), jnp.int32),
            pltpu.VMEM((nchunk, CB), jnp.int32),
            pltpu.VMEM((CB, h), jnp.float32),
            pltpu.VMEM_SHARED((npad, h), jnp.float32),
            pltpu.SemaphoreType.DMA,
        ],
    )
    def scatter_kernel(g_hbm, src_hbm, dst_hbm, zeros_hbm, out_hbm,
                       srcv, dstv, rows, acc, gsem):
        c = lax.axis_index("c")
        s = lax.axis_index("s")
        wid = s * NC + c
        r0 = s * rows_per_tile
        pltpu.sync_copy(zeros_hbm.at[pl.ds(r0, rows_per_tile)],
                        acc.at[pl.ds(r0, rows_per_tile)])
        pltpu.sync_copy(src_hbm.at[wid], srcv)
        pltpu.sync_copy(dst_hbm.at[wid], dstv)
        plsc.subcore_barrier()

        def body(j, carry):
            pltpu.async_copy(g_hbm.at[srcv.at[j]], rows, gsem).wait()
            pltpu.sync_copy(rows, acc.at[dstv.at[j]], add=True)
            return carry

        lax.fori_loop(0, nchunk, body, 0)
        plsc.subcore_barrier()
        pltpu.sync_copy(acc.at[pl.ds(r0, rows_per_tile)],
                        out_hbm.at[c, pl.ds(r0, rows_per_tile)])

    return scatter_kernel


def _tc_first(degp, x, w1, br):
    n, f = x.shape
    h = w1.shape[1]
    grid = n // br

    def body(deg_ref, x_ref, w_ref, dis_ref, g_ref):
        a = deg_ref[...]
        deg = a[0, :, 0:1] + a[1, :, 0:1] + 1.0
        dis = lax.rsqrt(deg)
        m = jnp.dot(x_ref[...], w_ref[...], preferred_element_type=jnp.float32)
        g_ref[...] = dis * m
        dis_ref[...] = jnp.broadcast_to(dis, (br, h))

    return pl.pallas_call(
        body,
        grid=(grid,),
        in_specs=[
            pl.BlockSpec((NC, br, DEGW), lambda i: (0, i, 0)),
            pl.BlockSpec((br, f), lambda i: (i, 0)),
            pl.BlockSpec((f, h), lambda i: (0, 0)),
        ],
        out_specs=[
            pl.BlockSpec((br, h), lambda i: (i, 0)),
            pl.BlockSpec((br, h), lambda i: (i, 0)),
        ],
        out_shape=[
            jax.ShapeDtypeStruct((n, h), jnp.float32),
            jax.ShapeDtypeStruct((n, h), jnp.float32),
        ],
    )(degp, x, w1)


def _tc_combine(agg, g, dis64, b, w_next, br):
    n, h = g.shape
    h2 = w_next.shape[1]
    grid = n // br

    def body(agg_ref, g_ref, dis_ref, b_ref, w_ref, out_ref):
        a = agg_ref[...]
        dis = dis_ref[...]
        hcur = jax.nn.relu(dis * (a[0] + a[1] + g_ref[...]) + b_ref[...])
        out_ref[...] = dis[:, :h2] * jnp.dot(
            hcur, w_ref[...], preferred_element_type=jnp.float32)

    return pl.pallas_call(
        body,
        grid=(grid,),
        in_specs=[
            pl.BlockSpec((NC, br, h), lambda i: (0, i, 0)),
            pl.BlockSpec((br, h), lambda i: (i, 0)),
            pl.BlockSpec((br, h), lambda i: (i, 0)),
            pl.BlockSpec((1, h), lambda i: (0, 0)),
            pl.BlockSpec((h, h2), lambda i: (0, 0)),
        ],
        out_specs=pl.BlockSpec((br, h2), lambda i: (i, 0)),
        out_shape=jax.ShapeDtypeStruct((n, h2), jnp.float32),
    )(agg, g, dis64, b, w_next)


def _tc_final(agg, g, dis64, b, wfc, bfc, br):
    n, h = g.shape
    grid = n // br

    def body(agg_ref, g_ref, dis_ref, b_ref, wfc_ref, bfc_ref, out_ref, acc_ref):
        i = pl.program_id(0)
        a = agg_ref[...]
        blk = dis_ref[...] * (a[0] + a[1] + g_ref[...]) + b_ref[...]
        s = jnp.sum(blk, axis=0, keepdims=True)

        @pl.when(i == 0)
        def _():
            acc_ref[...] = s

        @pl.when(i > 0)
        def _():
            acc_ref[...] += s

        @pl.when(i == grid - 1)
        def _():
            pooled = acc_ref[...] * (1.0 / n)
            out_ref[...] = jnp.dot(
                pooled, wfc_ref[...], preferred_element_type=jnp.float32
            ) + bfc_ref[...]

    return pl.pallas_call(
        body,
        grid=(grid,),
        in_specs=[
            pl.BlockSpec((NC, br, h), lambda i: (0, i, 0)),
            pl.BlockSpec((br, h), lambda i: (i, 0)),
            pl.BlockSpec((br, h), lambda i: (i, 0)),
            pl.BlockSpec((1, h), lambda i: (0, 0)),
            pl.BlockSpec((h, 1), lambda i: (0, 0)),
            pl.BlockSpec((1, 1), lambda i: (0, 0)),
        ],
        out_specs=pl.BlockSpec((1, 1), lambda i: (0, 0)),
        out_shape=jax.ShapeDtypeStruct((1, 1), jnp.float32),
        scratch_shapes=[pltpu.VMEM((1, h), jnp.float32)],
    )(agg, g, dis64, b, wfc, bfc)


def kernel(x, edge_index, W1, b1, W2, b2, W3, b3, Wfc, bfc):
    n, f = x.shape
    h = W1.shape[1]
    e = edge_index.shape[1]

    npad = ((n + NS * CB - 1) // (NS * CB)) * (NS * CB)
    epw = ((e + NW * CB - 1) // (NW * CB)) * CB
    epad = epw * NW
    nchunk = epw // CB
    pad_row = n + 8

    src = edge_index[0]
    dst = edge_index[1]
    pad = epad - e
    src_p = jnp.concatenate(
        [src, jnp.zeros((pad,), jnp.int32)]).reshape(NW, nchunk, CB)
    dst_p = jnp.concatenate(
        [dst, jnp.full((pad,), pad_row, jnp.int32)]).reshape(NW, nchunk, CB)

    ones_deg = jnp.ones((CB, DEGW), jnp.float32)
    zeros_deg = jnp.zeros((npad, DEGW), jnp.float32)
    zeros_agg = jnp.zeros((npad, h), jnp.float32)

    br = 1000

    degp = _sc_degree_fn(npad, nchunk)(dst_p, ones_deg, zeros_deg)
    dis64, g = _tc_first(degp, x, W1, br)

    scat = _sc_scatter_fn(n, h, npad, nchunk)
    agg = scat(g, src_p, dst_p, zeros_agg)
    g = _tc_combine(agg, g, dis64, b1.reshape(1, h), W2, br)
    agg = scat(g, src_p, dst_p, zeros_agg)
    g = _tc_combine(agg, g, dis64, b2.reshape(1, h), W3, br)
    agg = scat(g, src_p, dst_p, zeros_agg)
    return _tc_final(agg, g, dis64, b3.reshape(1, h), Wfc,
                     bfc.reshape(1, 1), br)


# speedup vs baseline: 25.3683x; 1.4309x over previous
import functools

import jax
import jax.numpy as jnp
from jax import lax
from jax.experimental import pallas as pl
from jax.experimental.pallas import tpu as pltpu
from jax.experimental.pallas import tpu_sc as plsc

NC = 2
NS = 16
NW = NC * NS
CB = 128

DEGW = 16

_MESH = plsc.VectorSubcoreMesh(
    core_axis_name="c", subcore_axis_name="s", num_cores=NC, num_subcores=NS
)

_SC_PARAMS = pltpu.CompilerParams(use_tc_tiling_on_sc=False)


def _sc_degree_fn(npad, nchunk):
    rows_per_tile = npad // NS

    @functools.partial(
        pl.kernel,
        out_type=jax.ShapeDtypeStruct((NC, npad, DEGW), jnp.float32),
        mesh=_MESH,
        compiler_params=_SC_PARAMS,
        scratch_types=[
            pltpu.VMEM((nchunk, CB), jnp.int32),
            pltpu.VMEM((CB, DEGW), jnp.float32),
            pltpu.VMEM_SHARED((npad, DEGW), jnp.float32),
        ],
    )
    def deg_kernel(dst_hbm, ones_hbm, zeros_hbm, out_hbm, dstv, onesv, acc):
        c = lax.axis_index("c")
        s = lax.axis_index("s")
        wid = s * NC + c
        r0 = s * rows_per_tile
        pltpu.sync_copy(zeros_hbm.at[pl.ds(r0, rows_per_tile)],
                        acc.at[pl.ds(r0, rows_per_tile)])
        pltpu.sync_copy(ones_hbm, onesv)
        pltpu.sync_copy(dst_hbm.at[wid], dstv)
        plsc.subcore_barrier()

        def body(j, carry):
            pltpu.sync_copy(onesv, acc.at[dstv.at[j]], add=True)
            return carry

        lax.fori_loop(0, nchunk, body, 0)
        plsc.subcore_barrier()
        pltpu.sync_copy(acc.at[pl.ds(r0, rows_per_tile)],
                        out_hbm.at[c, pl.ds(r0, rows_per_tile)])

    return deg_kernel


def _sc_scatter_fn(n, h, npad, nchunk):
    rows_per_tile = npad // NS

    @functools.partial(
        pl.kernel,
        out_type=jax.ShapeDtypeStruct((NC, npad, h), jnp.float32),
        mesh=_MESH,
        compiler_params=_SC_PARAMS,
        scratch_types=[
            pltpu.VMEM((nchunk, CB), jnp.int32),
            pltpu.VMEM((nchunk, CB), jnp.int32),
            pltpu.VMEM((CB, h), jnp.float32),
            pltpu.VMEM_SHARED((npad, h), jnp.float32),
            pltpu.VMEM_SHARED((npad, h), jnp.float32),
            pltpu.SemaphoreType.DMA,
        ],
    )
    def scatter_kernel(g_hbm, src_hbm, dst_hbm, zeros_hbm, out_hbm,
                       srcv, dstv, rows, gtab, acc, gsem):
        c = lax.axis_index("c")
        s = lax.axis_index("s")
        wid = s * NC + c
        r0 = s * rows_per_tile
        pltpu.sync_copy(g_hbm.at[pl.ds(r0, rows_per_tile)],
                        gtab.at[pl.ds(r0, rows_per_tile)])
        pltpu.sync_copy(zeros_hbm.at[pl.ds(r0, rows_per_tile)],
                        acc.at[pl.ds(r0, rows_per_tile)])
        pltpu.sync_copy(src_hbm.at[wid], srcv)
        pltpu.sync_copy(dst_hbm.at[wid], dstv)
        plsc.subcore_barrier()

        def body(j, carry):
            pltpu.async_copy(gtab.at[srcv.at[j]], rows, gsem).wait()
            pltpu.sync_copy(rows, acc.at[dstv.at[j]], add=True)
            return carry

        lax.fori_loop(0, nchunk, body, 0)
        plsc.subcore_barrier()
        pltpu.sync_copy(acc.at[pl.ds(r0, rows_per_tile)],
                        out_hbm.at[c, pl.ds(r0, rows_per_tile)])

    return scatter_kernel


def _tc_first(degp, x, w1, br):
    n, f = x.shape
    h = w1.shape[1]
    grid = n // br

    def body(deg_ref, x_ref, w_ref, dis_ref, g_ref):
        a = deg_ref[...]
        deg = a[0, :, 0:1] + a[1, :, 0:1] + 1.0
        dis = lax.rsqrt(deg)
        m = jnp.dot(x_ref[...], w_ref[...], preferred_element_type=jnp.float32)
        g_ref[...] = dis * m
        dis_ref[...] = jnp.broadcast_to(dis, (br, h))

    return pl.pallas_call(
        body,
        grid=(grid,),
        in_specs=[
            pl.BlockSpec((NC, br, DEGW), lambda i: (0, i, 0)),
            pl.BlockSpec((br, f), lambda i: (i, 0)),
            pl.BlockSpec((f, h), lambda i: (0, 0)),
        ],
        out_specs=[
            pl.BlockSpec((br, h), lambda i: (i, 0)),
            pl.BlockSpec((br, h), lambda i: (i, 0)),
        ],
        out_shape=[
            jax.ShapeDtypeStruct((n, h), jnp.float32),
            jax.ShapeDtypeStruct((n, h), jnp.float32),
        ],
    )(degp, x, w1)


def _tc_combine(agg, g, dis64, b, w_next, br):
    n, h = g.shape
    h2 = w_next.shape[1]
    grid = n // br

    def body(agg_ref, g_ref, dis_ref, b_ref, w_ref, out_ref):
        a = agg_ref[...]
        dis = dis_ref[...]
        hcur = jax.nn.relu(dis * (a[0] + a[1] + g_ref[...]) + b_ref[...])
        out_ref[...] = dis[:, :h2] * jnp.dot(
            hcur, w_ref[...], preferred_element_type=jnp.float32)

    return pl.pallas_call(
        body,
        grid=(grid,),
        in_specs=[
            pl.BlockSpec((NC, br, h), lambda i: (0, i, 0)),
            pl.BlockSpec((br, h), lambda i: (i, 0)),
            pl.BlockSpec((br, h), lambda i: (i, 0)),
            pl.BlockSpec((1, h), lambda i: (0, 0)),
            pl.BlockSpec((h, h2), lambda i: (0, 0)),
        ],
        out_specs=pl.BlockSpec((br, h2), lambda i: (i, 0)),
        out_shape=jax.ShapeDtypeStruct((n, h2), jnp.float32),
    )(agg, g, dis64, b, w_next)


def _tc_final(agg, g, dis64, b, wfc, bfc, br):
    n, h = g.shape
    grid = n // br

    def body(agg_ref, g_ref, dis_ref, b_ref, wfc_ref, bfc_ref, out_ref, acc_ref):
        i = pl.program_id(0)
        a = agg_ref[...]
        blk = dis_ref[...] * (a[0] + a[1] + g_ref[...]) + b_ref[...]
        s = jnp.sum(blk, axis=0, keepdims=True)

        @pl.when(i == 0)
        def _():
            acc_ref[...] = s

        @pl.when(i > 0)
        def _():
            acc_ref[...] += s

        @pl.when(i == grid - 1)
        def _():
            pooled = acc_ref[...] * (1.0 / n)
            out_ref[...] = jnp.dot(
                pooled, wfc_ref[...], preferred_element_type=jnp.float32
            ) + bfc_ref[...]

    return pl.pallas_call(
        body,
        grid=(grid,),
        in_specs=[
            pl.BlockSpec((NC, br, h), lambda i: (0, i, 0)),
            pl.BlockSpec((br, h), lambda i: (i, 0)),
            pl.BlockSpec((br, h), lambda i: (i, 0)),
            pl.BlockSpec((1, h), lambda i: (0, 0)),
            pl.BlockSpec((h, 1), lambda i: (0, 0)),
            pl.BlockSpec((1, 1), lambda i: (0, 0)),
        ],
        out_specs=pl.BlockSpec((1, 1), lambda i: (0, 0)),
        out_shape=jax.ShapeDtypeStruct((1, 1), jnp.float32),
        scratch_shapes=[pltpu.VMEM((1, h), jnp.float32)],
    )(agg, g, dis64, b, wfc, bfc)


def kernel(x, edge_index, W1, b1, W2, b2, W3, b3, Wfc, bfc):
    n, f = x.shape
    h = W1.shape[1]
    e = edge_index.shape[1]

    npad = ((n + NS * CB - 1) // (NS * CB)) * (NS * CB)
    epw = ((e + NW * CB - 1) // (NW * CB)) * CB
    epad = epw * NW
    nchunk = epw // CB
    pad_row = n + 8

    src = edge_index[0]
    dst = edge_index[1]
    pad = epad - e
    src_p = jnp.concatenate(
        [src, jnp.zeros((pad,), jnp.int32)]).reshape(NW, nchunk, CB)
    dst_p = jnp.concatenate(
        [dst, jnp.full((pad,), pad_row, jnp.int32)]).reshape(NW, nchunk, CB)

    ones_deg = jnp.ones((CB, DEGW), jnp.float32)
    zeros_deg = jnp.zeros((npad, DEGW), jnp.float32)
    zeros_agg = jnp.zeros((npad, h), jnp.float32)

    br = 1000

    degp = _sc_degree_fn(npad, nchunk)(dst_p, ones_deg, zeros_deg)
    dis64, g = _tc_first(degp, x, W1, br)

    scat = _sc_scatter_fn(n, h, npad, nchunk)
    gpad = ((0, npad - n), (0, 0))
    agg = scat(jnp.pad(g, gpad), src_p, dst_p, zeros_agg)
    g = _tc_combine(agg, g, dis64, b1.reshape(1, h), W2, br)
    agg = scat(jnp.pad(g, gpad), src_p, dst_p, zeros_agg)
    g = _tc_combine(agg, g, dis64, b2.reshape(1, h), W3, br)
    agg = scat(jnp.pad(g, gpad), src_p, dst_p, zeros_agg)
    return _tc_final(agg, g, dis64, b3.reshape(1, h), Wfc,
                     bfc.reshape(1, 1), br)


# speedup vs baseline: 27.6869x; 1.0914x over previous
import functools

import jax
import jax.numpy as jnp
from jax import lax
from jax.experimental import pallas as pl
from jax.experimental.pallas import tpu as pltpu
from jax.experimental.pallas import tpu_sc as plsc

NC = 2
NS = 16
NW = NC * NS
CB = 128

DEGW = 16

_MESH = plsc.VectorSubcoreMesh(
    core_axis_name="c", subcore_axis_name="s", num_cores=NC, num_subcores=NS
)

_SC_PARAMS = pltpu.CompilerParams(use_tc_tiling_on_sc=False)


def _sc_degree_fn(npad, nchunk):
    rows_per_tile = npad // NS

    @functools.partial(
        pl.kernel,
        out_type=jax.ShapeDtypeStruct((NC, npad, DEGW), jnp.float32),
        mesh=_MESH,
        compiler_params=_SC_PARAMS,
        scratch_types=[
            pltpu.VMEM((nchunk, CB), jnp.int32),
            pltpu.VMEM((CB, DEGW), jnp.float32),
            pltpu.VMEM_SHARED((npad, DEGW), jnp.float32),
        ],
    )
    def deg_kernel(dst_hbm, ones_hbm, zeros_hbm, out_hbm, dstv, onesv, acc):
        c = lax.axis_index("c")
        s = lax.axis_index("s")
        wid = s * NC + c
        r0 = s * rows_per_tile
        pltpu.sync_copy(zeros_hbm.at[pl.ds(r0, rows_per_tile)],
                        acc.at[pl.ds(r0, rows_per_tile)])
        pltpu.sync_copy(ones_hbm, onesv)
        pltpu.sync_copy(dst_hbm.at[wid], dstv)
        plsc.subcore_barrier()

        def body(j, carry):
            pltpu.sync_copy(onesv, acc.at[dstv.at[j]], add=True)
            return carry

        lax.fori_loop(0, nchunk, body, 0)
        plsc.subcore_barrier()
        pltpu.sync_copy(acc.at[pl.ds(r0, rows_per_tile)],
                        out_hbm.at[c, pl.ds(r0, rows_per_tile)])

    return deg_kernel


def _sc_scatter_fn(n, h, npad, nchunk):
    rows_per_tile = npad // NS

    @functools.partial(
        pl.kernel,
        out_type=jax.ShapeDtypeStruct((NC, npad, h), jnp.float32),
        mesh=_MESH,
        compiler_params=_SC_PARAMS,
        scratch_types=[
            pltpu.VMEM((nchunk, CB), jnp.int32),
            pltpu.VMEM((nchunk, CB), jnp.int32),
            pltpu.VMEM((CB, h), jnp.float32),
            pltpu.VMEM((CB, h), jnp.float32),
            pltpu.VMEM_SHARED((npad, h), jnp.float32),
            pltpu.VMEM_SHARED((npad, h), jnp.float32),
            pltpu.SemaphoreType.DMA,
            pltpu.SemaphoreType.DMA,
            pltpu.SemaphoreType.DMA,
            pltpu.SemaphoreType.DMA,
        ],
    )
    def scatter_kernel(g_hbm, src_hbm, dst_hbm, zeros_hbm, out_hbm,
                       srcv, dstv, rows0, rows1, gtab, acc,
                       sem0, sem1, sem2, sem3):
        c = lax.axis_index("c")
        s = lax.axis_index("s")
        wid = s * NC + c
        r0 = s * rows_per_tile
        cp_g = pltpu.async_copy(g_hbm.at[pl.ds(r0, rows_per_tile)],
                                gtab.at[pl.ds(r0, rows_per_tile)], sem0)
        cp_z = pltpu.async_copy(zeros_hbm.at[pl.ds(r0, rows_per_tile)],
                                acc.at[pl.ds(r0, rows_per_tile)], sem1)
        cp_s = pltpu.async_copy(src_hbm.at[wid], srcv, sem2)
        cp_d = pltpu.async_copy(dst_hbm.at[wid], dstv, sem3)
        cp_g.wait()
        cp_z.wait()
        cp_s.wait()
        cp_d.wait()
        plsc.subcore_barrier()

        def body(i, carry):
            j0 = 2 * i
            j1 = j0 + 1
            pltpu.async_copy(gtab.at[srcv.at[j0]], rows0, sem0).wait()
            cp1 = pltpu.async_copy(gtab.at[srcv.at[j1]], rows1, sem1)
            pltpu.sync_copy(rows0, acc.at[dstv.at[j0]], add=True)
            cp1.wait()
            pltpu.sync_copy(rows1, acc.at[dstv.at[j1]], add=True)
            return carry

        lax.fori_loop(0, nchunk // 2, body, 0)
        if nchunk % 2:
            j = nchunk - 1
            pltpu.async_copy(gtab.at[srcv.at[j]], rows0, sem0).wait()
            pltpu.sync_copy(rows0, acc.at[dstv.at[j]], add=True)
        plsc.subcore_barrier()
        pltpu.sync_copy(acc.at[pl.ds(r0, rows_per_tile)],
                        out_hbm.at[c, pl.ds(r0, rows_per_tile)])

    return scatter_kernel


def _tc_first(degp, x, w1, br):
    n, f = x.shape
    h = w1.shape[1]
    grid = n // br

    def body(deg_ref, x_ref, w_ref, dis_ref, g_ref):
        a = deg_ref[...]
        deg = a[0, :, 0:1] + a[1, :, 0:1] + 1.0
        dis = lax.rsqrt(deg)
        m = jnp.dot(x_ref[...], w_ref[...], preferred_element_type=jnp.float32)
        g_ref[...] = dis * m
        dis_ref[...] = jnp.broadcast_to(dis, (br, h))

    return pl.pallas_call(
        body,
        grid=(grid,),
        in_specs=[
            pl.BlockSpec((NC, br, DEGW), lambda i: (0, i, 0)),
            pl.BlockSpec((br, f), lambda i: (i, 0)),
            pl.BlockSpec((f, h), lambda i: (0, 0)),
        ],
        out_specs=[
            pl.BlockSpec((br, h), lambda i: (i, 0)),
            pl.BlockSpec((br, h), lambda i: (i, 0)),
        ],
        out_shape=[
            jax.ShapeDtypeStruct((n, h), jnp.float32),
            jax.ShapeDtypeStruct((n, h), jnp.float32),
        ],
    )(degp, x, w1)


def _tc_combine(agg, g, dis64, b, w_next, br):
    n, h = g.shape
    h2 = w_next.shape[1]
    grid = n // br

    def body(agg_ref, g_ref, dis_ref, b_ref, w_ref, out_ref):
        a = agg_ref[...]
        dis = dis_ref[...]
        hcur = jax.nn.relu(dis * (a[0] + a[1] + g_ref[...]) + b_ref[...])
        out_ref[...] = dis[:, :h2] * jnp.dot(
            hcur, w_ref[...], preferred_element_type=jnp.float32)

    return pl.pallas_call(
        body,
        grid=(grid,),
        in_specs=[
            pl.BlockSpec((NC, br, h), lambda i: (0, i, 0)),
            pl.BlockSpec((br, h), lambda i: (i, 0)),
            pl.BlockSpec((br, h), lambda i: (i, 0)),
            pl.BlockSpec((1, h), lambda i: (0, 0)),
            pl.BlockSpec((h, h2), lambda i: (0, 0)),
        ],
        out_specs=pl.BlockSpec((br, h2), lambda i: (i, 0)),
        out_shape=jax.ShapeDtypeStruct((n, h2), jnp.float32),
    )(agg, g, dis64, b, w_next)


def _tc_final(agg, g, dis64, b, wfc, bfc, br):
    n, h = g.shape
    grid = n // br

    def body(agg_ref, g_ref, dis_ref, b_ref, wfc_ref, bfc_ref, out_ref, acc_ref):
        i = pl.program_id(0)
        a = agg_ref[...]
        blk = dis_ref[...] * (a[0] + a[1] + g_ref[...]) + b_ref[...]
        s = jnp.sum(blk, axis=0, keepdims=True)

        @pl.when(i == 0)
        def _():
            acc_ref[...] = s

        @pl.when(i > 0)
        def _():
            acc_ref[...] += s

        @pl.when(i == grid - 1)
        def _():
            pooled = acc_ref[...] * (1.0 / n)
            out_ref[...] = jnp.dot(
                pooled, wfc_ref[...], preferred_element_type=jnp.float32
            ) + bfc_ref[...]

    return pl.pallas_call(
        body,
        grid=(grid,),
        in_specs=[
            pl.BlockSpec((NC, br, h), lambda i: (0, i, 0)),
            pl.BlockSpec((br, h), lambda i: (i, 0)),
            pl.BlockSpec((br, h), lambda i: (i, 0)),
            pl.BlockSpec((1, h), lambda i: (0, 0)),
            pl.BlockSpec((h, 1), lambda i: (0, 0)),
            pl.BlockSpec((1, 1), lambda i: (0, 0)),
        ],
        out_specs=pl.BlockSpec((1, 1), lambda i: (0, 0)),
        out_shape=jax.ShapeDtypeStruct((1, 1), jnp.float32),
        scratch_shapes=[pltpu.VMEM((1, h), jnp.float32)],
    )(agg, g, dis64, b, wfc, bfc)


def kernel(x, edge_index, W1, b1, W2, b2, W3, b3, Wfc, bfc):
    n, f = x.shape
    h = W1.shape[1]
    e = edge_index.shape[1]

    npad = ((n + NS * CB - 1) // (NS * CB)) * (NS * CB)
    epw = ((e + NW * CB - 1) // (NW * CB)) * CB
    epad = epw * NW
    nchunk = epw // CB
    pad_row = n + 8

    src = edge_index[0]
    dst = edge_index[1]
    pad = epad - e
    src_p = jnp.concatenate(
        [src, jnp.zeros((pad,), jnp.int32)]).reshape(NW, nchunk, CB)
    dst_p = jnp.concatenate(
        [dst, jnp.full((pad,), pad_row, jnp.int32)]).reshape(NW, nchunk, CB)

    ones_deg = jnp.ones((CB, DEGW), jnp.float32)
    zeros_deg = jnp.zeros((npad, DEGW), jnp.float32)
    zeros_agg = jnp.zeros((npad, h), jnp.float32)

    br = 1000

    degp = _sc_degree_fn(npad, nchunk)(dst_p, ones_deg, zeros_deg)
    dis64, g = _tc_first(degp, x, W1, br)

    scat = _sc_scatter_fn(n, h, npad, nchunk)
    gpad = ((0, npad - n), (0, 0))
    agg = scat(jnp.pad(g, gpad), src_p, dst_p, zeros_agg)
    g = _tc_combine(agg, g, dis64, b1.reshape(1, h), W2, br)
    agg = scat(jnp.pad(g, gpad), src_p, dst_p, zeros_agg)
    g = _tc_combine(agg, g, dis64, b2.reshape(1, h), W3, br)
    agg = scat(jnp.pad(g, gpad), src_p, dst_p, zeros_agg)
    return _tc_final(agg, g, dis64, b3.reshape(1, h), Wfc,
                     bfc.reshape(1, 1), br)
